# baseline (speedup 1.0000x reference)
import functools

import jax
import jax.numpy as jnp
from jax import lax
from jax.experimental import pallas as pl
from jax.experimental.pallas import tpu as pltpu
from jax.experimental.pallas import tpu_sc as plsc

_HID = 64
_L = 200
_B = 4096
_VOCAB = 1000000
_NW = 32
_ROWS_PER_W = _B // _NW
_NBUF = 2
_SPLITS = ((0, 104), (104, 96))


def _tpe_body(x_hbm, tok_hbm, pos_hbm, out_hbm, *scratch):
  bufs = scratch[0:_NBUF]
  idxs = scratch[_NBUF:2 * _NBUF]
  obufs = scratch[2 * _NBUF:3 * _NBUF]
  hsels = scratch[3 * _NBUF:4 * _NBUF]
  pos_v = scratch[4 * _NBUF]
  isems = scratch[4 * _NBUF + 1:4 * _NBUF + 1 + _NBUF]
  gsems = scratch[4 * _NBUF + 1 + _NBUF:4 * _NBUF + 1 + 2 * _NBUF]
  ssems = scratch[4 * _NBUF + 1 + 2 * _NBUF:]

  wid = lax.axis_index("s") * 2 + lax.axis_index("c")
  row0 = wid * _ROWS_PER_W

  pltpu.sync_copy(pos_hbm.at[pl.ds(0, _L)], pos_v)

  def start_idx(ci, s):
    base = (row0 + ci) * _L
    pltpu.make_async_copy(
        x_hbm.at[pl.ds(base, _L)], idxs[s].at[pl.ds(0, _L)],
        isems[s]).start()

  def prep_and_gather(s):
    pltpu.make_async_copy(
        x_hbm.at[pl.ds(0, _L)], idxs[s].at[pl.ds(0, _L)],
        isems[s]).wait()

    @plsc.parallel_loop(0, _L, 16)
    def _(r):
      sl = pl.ds(r, 16)
      v = idxs[s][sl]
      hsels[s][sl] = (v & 1) * _HID
      idxs[s][sl] = jax.lax.shift_right_logical(v, 1)

    for (off, n) in _SPLITS:
      pltpu.make_async_copy(
          tok_hbm.at[idxs[s].at[pl.ds(off, n)]],
          bufs[s].at[pl.ds(off, n)],
          gsems[s],
      ).start()

  def wait_gather(s):
    pltpu.make_async_copy(
        tok_hbm.at[idxs[s].at[pl.ds(0, _L)]], bufs[s], gsems[s]).wait()

  def start_scatter(ci, s):
    pltpu.make_async_copy(
        obufs[s], out_hbm.at[row0 + ci], ssems[s]).start()

  def wait_scatter(s):
    pltpu.make_async_copy(
        obufs[s], out_hbm.at[0], ssems[s]).wait()

  def add_pos(s):
    buf = bufs[s]
    obuf = obufs[s]
    hsel = hsels[s]

    def rows16(g, pv, lanes):
      for k in lanes:
        h64 = pv[k]
        r = g + k
        for c in range(_HID // 16):
          obuf[r, pl.ds(c * 16, 16)] = (
              buf[r, pl.ds(h64 + c * 16, 16)] + pos_v[r, pl.ds(c * 16, 16)])

    @plsc.parallel_loop(0, _L - 8, 16)
    def _(g):
      rows16(g, hsel[pl.ds(g, 16)], range(16))

    rows16(_L - 16, hsel[pl.ds(_L - 16, 16)], range(8, 16))

  start_idx(0, 0)
  start_idx(1, 1)
  prep_and_gather(0)

  def step(i, carry):
    for k in range(_NBUF):
      ci = i * _NBUF + k

      @pl.when(ci + 1 < _ROWS_PER_W)
      def _():
        prep_and_gather(1 - k)

      @pl.when(ci >= 2)
      def _():
        wait_scatter(k)

      wait_gather(k)

      @pl.when(ci + 2 < _ROWS_PER_W)
      def _():
        start_idx(ci + 2, k)

      add_pos(k)
      start_scatter(ci, k)
    return carry

  lax.fori_loop(0, _ROWS_PER_W // _NBUF, step, 0)

  for s in range(_NBUF):
    wait_scatter(s)


@jax.jit
def _tpe_call(x_flat, tok_pair, pos_table):
  mesh = plsc.VectorSubcoreMesh(core_axis_name="c", subcore_axis_name="s")
  kern = functools.partial(
      pl.kernel,
      mesh=mesh,
      compiler_params=pltpu.CompilerParams(use_tc_tiling_on_sc=False),
      out_type=jax.ShapeDtypeStruct((_B, _L, _HID), jnp.float32),
      scratch_types=(
          [pltpu.VMEM((_L, 2 * _HID), jnp.float32) for _ in range(_NBUF)]
          + [pltpu.VMEM((208,), jnp.int32) for _ in range(_NBUF)]
          + [pltpu.VMEM((_L, _HID), jnp.float32) for _ in range(_NBUF)]
          + [pltpu.VMEM((208,), jnp.int32) for _ in range(_NBUF)]
          + [pltpu.VMEM((_L, _HID), jnp.float32)]
          + [pltpu.SemaphoreType.DMA] * (3 * _NBUF)
      ),
  )(_tpe_body)
  return kern(x_flat, tok_pair, pos_table)


def kernel(x, tok_table, pos_table):
  x_flat = jnp.reshape(x.astype(jnp.int32), (_B * _L,))
  tok_pair = jnp.reshape(tok_table, (_VOCAB // 2, 2 * _HID))
  return _tpe_call(x_flat, tok_pair, pos_table)


# speedup vs baseline: 1.3595x; 1.3595x over previous
import functools

import jax
import jax.numpy as jnp
from jax import lax
from jax.experimental import pallas as pl
from jax.experimental.pallas import tpu as pltpu
from jax.experimental.pallas import tpu_sc as plsc

_HID = 64
_L = 200
_B = 4096
_NW = 32
_ROWS_PER_W = _B // _NW
_NBUF = 2
_SPLITS = ((0, 104), (104, 96))


def _tpe_body(x_hbm, tok_hbm, pos_hbm, out_hbm, *scratch):
  bufs = scratch[0:_NBUF]
  idxs = scratch[_NBUF:2 * _NBUF]
  obufs = scratch[2 * _NBUF:3 * _NBUF]
  pos_v = scratch[3 * _NBUF]
  isems = scratch[3 * _NBUF + 1:3 * _NBUF + 1 + _NBUF]
  gsems = scratch[3 * _NBUF + 1 + _NBUF:3 * _NBUF + 1 + 2 * _NBUF]
  ssems = scratch[3 * _NBUF + 1 + 2 * _NBUF:]

  wid = lax.axis_index("s") * 2 + lax.axis_index("c")
  row0 = wid * _ROWS_PER_W

  pltpu.sync_copy(pos_hbm.at[pl.ds(0, _L)], pos_v)

  def start_idx(ci, s):
    base = (row0 + ci) * _L
    pltpu.make_async_copy(
        x_hbm.at[pl.ds(base, _L)], idxs[s], isems[s]).start()

  def prep_and_gather(s):
    pltpu.make_async_copy(
        x_hbm.at[pl.ds(0, _L)], idxs[s], isems[s]).wait()
    for (off, n) in _SPLITS:
      pltpu.make_async_copy(
          tok_hbm.at[idxs[s].at[pl.ds(off, n)]],
          bufs[s].at[pl.ds(off, n)],
          gsems[s],
      ).start()

  def wait_gather(s):
    pltpu.make_async_copy(
        tok_hbm.at[idxs[s]], bufs[s], gsems[s]).wait()

  def start_scatter(ci, s):
    pltpu.make_async_copy(
        obufs[s], out_hbm.at[row0 + ci], ssems[s]).start()

  def wait_scatter(s):
    pltpu.make_async_copy(
        obufs[s], out_hbm.at[0], ssems[s]).wait()

  def add_pos(s):
    buf = bufs[s]
    obuf = obufs[s]

    @plsc.parallel_loop(0, _L, 1, unroll=4)
    def _(r):
      for c in range(_HID // 16):
        sl = pl.ds(c * 16, 16)
        obuf[r, sl] = buf[r, sl] + pos_v[r, sl]

  start_idx(0, 0)
  start_idx(1, 1)
  prep_and_gather(0)

  def step(i, carry):
    for k in range(_NBUF):
      ci = i * _NBUF + k

      @pl.when(ci + 1 < _ROWS_PER_W)
      def _():
        prep_and_gather(1 - k)

      @pl.when(ci >= 2)
      def _():
        wait_scatter(k)

      wait_gather(k)

      @pl.when(ci + 2 < _ROWS_PER_W)
      def _():
        start_idx(ci + 2, k)

      add_pos(k)
      start_scatter(ci, k)
    return carry

  lax.fori_loop(0, _ROWS_PER_W // _NBUF, step, 0)

  for s in range(_NBUF):
    wait_scatter(s)


@jax.jit
def _tpe_call(x_flat, tok_table, pos_table):
  mesh = plsc.VectorSubcoreMesh(core_axis_name="c", subcore_axis_name="s")
  kern = functools.partial(
      pl.kernel,
      mesh=mesh,
      compiler_params=pltpu.CompilerParams(use_tc_tiling_on_sc=False),
      out_type=jax.ShapeDtypeStruct((_B, _L, 2 * _HID), jnp.float32),
      scratch_types=(
          [pltpu.VMEM((_L, _HID), jnp.float32) for _ in range(_NBUF)]
          + [pltpu.VMEM((_L,), jnp.int32) for _ in range(_NBUF)]
          + [pltpu.VMEM((_L, 2 * _HID), jnp.float32) for _ in range(_NBUF)]
          + [pltpu.VMEM((_L, _HID), jnp.float32)]
          + [pltpu.SemaphoreType.DMA] * (3 * _NBUF)
      ),
  )(_tpe_body)
  return kern(x_flat, tok_table, pos_table)


def kernel(x, tok_table, pos_table):
  x_flat = jnp.reshape(x.astype(jnp.int32), (_B * _L,))
  out_wide = _tpe_call(x_flat, tok_table, pos_table)
  return out_wide[:, :, :_HID]


# speedup vs baseline: 1.3679x; 1.0062x over previous
import functools

import jax
import jax.numpy as jnp
from jax import lax
from jax.experimental import pallas as pl
from jax.experimental.pallas import tpu as pltpu
from jax.experimental.pallas import tpu_sc as plsc

_HID = 64
_L = 200
_B = 4096
_NW = 32
_ROWS_PER_W = _B // _NW
_NBUF = 2
_SPLITS = ((0, 104), (104, 96))


def _tpe_body(x_hbm, tok_hbm, pos_hbm, out_hbm, *scratch):
  obufs = scratch[0:_NBUF]
  idxs = scratch[_NBUF:2 * _NBUF]
  pos_v = scratch[2 * _NBUF]
  isems = scratch[2 * _NBUF + 1:2 * _NBUF + 1 + _NBUF]
  gsems = scratch[2 * _NBUF + 1 + _NBUF:2 * _NBUF + 1 + 2 * _NBUF]
  ssems = scratch[2 * _NBUF + 1 + 2 * _NBUF:]

  wid = lax.axis_index("s") * 2 + lax.axis_index("c")
  row0 = wid * _ROWS_PER_W

  pltpu.sync_copy(pos_hbm.at[pl.ds(0, _L)], pos_v)

  def start_idx(ci, s):
    base = (row0 + ci) * _L
    pltpu.make_async_copy(
        x_hbm.at[pl.ds(base, _L)], idxs[s], isems[s]).start()

  def prep_and_gather(s):
    pltpu.make_async_copy(
        x_hbm.at[pl.ds(0, _L)], idxs[s], isems[s]).wait()
    for (off, n) in _SPLITS:
      pltpu.make_async_copy(
          tok_hbm.at[idxs[s].at[pl.ds(off, n)]],
          obufs[s].at[pl.ds(off, n)],
          gsems[s],
      ).start()

  def wait_gather(s):
    pltpu.make_async_copy(
        tok_hbm.at[idxs[s]], obufs[s], gsems[s]).wait()

  def start_scatter(ci, s):
    pltpu.make_async_copy(
        obufs[s], out_hbm.at[row0 + ci], ssems[s]).start()

  def wait_scatter(s):
    pltpu.make_async_copy(
        obufs[s], out_hbm.at[0], ssems[s]).wait()

  def add_pos(s):
    obuf = obufs[s]

    @plsc.parallel_loop(0, _L, 1, unroll=4)
    def _(r):
      for c in range(_HID // 16):
        sl = pl.ds(c * 16, 16)
        obuf[r, sl] = obuf[r, sl] + pos_v[r, sl]

  start_idx(0, 0)
  start_idx(1, 1)
  prep_and_gather(0)

  def step(i, carry):
    for k in range(_NBUF):
      ci = i * _NBUF + k

      @pl.when(ci + 1 < _ROWS_PER_W)
      def _():
        @pl.when(ci >= 1)
        def _():
          wait_scatter(1 - k)
        prep_and_gather(1 - k)

      wait_gather(k)

      @pl.when(ci + 2 < _ROWS_PER_W)
      def _():
        start_idx(ci + 2, k)

      add_pos(k)
      start_scatter(ci, k)
    return carry

  lax.fori_loop(0, _ROWS_PER_W // _NBUF, step, 0)

  for s in range(_NBUF):
    wait_scatter(s)


@jax.jit
def _tpe_call(x_flat, tok_wide, pos_table):
  mesh = plsc.VectorSubcoreMesh(core_axis_name="c", subcore_axis_name="s")
  kern = functools.partial(
      pl.kernel,
      mesh=mesh,
      compiler_params=pltpu.CompilerParams(use_tc_tiling_on_sc=False),
      out_type=jax.ShapeDtypeStruct((_B, _L, 2 * _HID), jnp.float32),
      scratch_types=(
          [pltpu.VMEM((_L, 2 * _HID), jnp.float32) for _ in range(_NBUF)]
          + [pltpu.VMEM((_L,), jnp.int32) for _ in range(_NBUF)]
          + [pltpu.VMEM((_L, _HID), jnp.float32)]
          + [pltpu.SemaphoreType.DMA] * (3 * _NBUF)
      ),
  )(_tpe_body)
  return kern(x_flat, tok_wide, pos_table)


def kernel(x, tok_table, pos_table):
  x_flat = jnp.reshape(x.astype(jnp.int32), (_B * _L,))
  tok_wide = jnp.pad(tok_table, ((0, 0), (0, _HID)))
  out_wide = _tpe_call(x_flat, tok_wide, pos_table)
  return out_wide[:, :, :_HID]


# speedup vs baseline: 1.5129x; 1.1060x over previous
import functools

import jax
import jax.numpy as jnp
from jax import lax
from jax.experimental import pallas as pl
from jax.experimental.pallas import tpu as pltpu
from jax.experimental.pallas import tpu_sc as plsc

_HID = 64
_L = 200
_B = 4096
_VOCAB = 1000000
_NW = 32
_ROWS_PER_W = _B // _NW
_NBUF = 2
_SPLITS = ((0, 104), (104, 96))


def _tpe_body(x_hbm, tok_hbm, pos_hbm, out_hbm, *scratch):
  bufs = scratch[0:_NBUF]
  idxs = scratch[_NBUF:2 * _NBUF]
  pos_v = scratch[2 * _NBUF]
  isems = scratch[2 * _NBUF + 1:2 * _NBUF + 1 + _NBUF]
  gsems = scratch[2 * _NBUF + 1 + _NBUF:2 * _NBUF + 1 + 2 * _NBUF]
  ssems = scratch[2 * _NBUF + 1 + 2 * _NBUF:]

  wid = lax.axis_index("s") * 2 + lax.axis_index("c")
  row0 = wid * _ROWS_PER_W

  pltpu.sync_copy(pos_hbm.at[pl.ds(0, _L)], pos_v)

  def start_idx(ci, s):
    base = (row0 + ci) * _L
    pltpu.make_async_copy(
        x_hbm.at[pl.ds(base, _L)], idxs[s].at[pl.ds(0, _L)],
        isems[s]).start()

  def prep_and_gather(s):
    pltpu.make_async_copy(
        x_hbm.at[pl.ds(0, _L)], idxs[s].at[pl.ds(0, _L)],
        isems[s]).wait()

    @plsc.parallel_loop(0, _L, 16)
    def _(r):
      sl = pl.ds(r, 16)
      idxs[s][sl] = idxs[s][sl] * 2

    for (off, n) in _SPLITS:
      pltpu.make_async_copy(
          tok_hbm.at[idxs[s].at[pl.ds(off, n)]],
          bufs[s].at[pl.ds(off, n)],
          gsems[s],
      ).start()

  def wait_gather(s):
    pltpu.make_async_copy(
        tok_hbm.at[idxs[s].at[pl.ds(0, _L)]], bufs[s], gsems[s]).wait()

  def start_scatter(ci, s):
    pltpu.make_async_copy(
        bufs[s], out_hbm.at[row0 + ci, pl.ds(0, _L), pl.ds(0, _HID)],
        ssems[s]).start()

  def wait_scatter(s):
    pltpu.make_async_copy(
        bufs[s], out_hbm.at[0, pl.ds(0, _L), pl.ds(0, _HID)],
        ssems[s]).wait()

  def add_pos(s):
    buf = bufs[s]

    @plsc.parallel_loop(0, _L, 1, unroll=4)
    def _(r):
      for c in range(_HID // 16):
        sl = pl.ds(c * 16, 16)
        buf[r, sl] = buf[r, sl] + pos_v[r, sl]

  start_idx(0, 0)
  start_idx(1, 1)
  prep_and_gather(0)

  def step(i, carry):
    for k in range(_NBUF):
      ci = i * _NBUF + k

      @pl.when(ci + 1 < _ROWS_PER_W)
      def _():
        @pl.when(ci >= 1)
        def _():
          wait_scatter(1 - k)
        prep_and_gather(1 - k)

      wait_gather(k)

      @pl.when(ci + 2 < _ROWS_PER_W)
      def _():
        start_idx(ci + 2, k)

      add_pos(k)
      start_scatter(ci, k)
    return carry

  lax.fori_loop(0, _ROWS_PER_W // _NBUF, step, 0)

  for s in range(_NBUF):
    wait_scatter(s)


@jax.jit
def _tpe_call(x_flat, tok_rows, pos_table):
  mesh = plsc.VectorSubcoreMesh(core_axis_name="c", subcore_axis_name="s")
  kern = functools.partial(
      pl.kernel,
      mesh=mesh,
      compiler_params=pltpu.CompilerParams(use_tc_tiling_on_sc=False),
      out_type=jax.ShapeDtypeStruct((_B, _L, 2 * _HID), jnp.float32),
      scratch_types=(
          [pltpu.VMEM((_L, _HID), jnp.float32) for _ in range(_NBUF)]
          + [pltpu.VMEM((208,), jnp.int32) for _ in range(_NBUF)]
          + [pltpu.VMEM((_L, _HID), jnp.float32)]
          + [pltpu.SemaphoreType.DMA] * (3 * _NBUF)
      ),
  )(_tpe_body)
  return kern(x_flat, tok_rows, pos_table)


def kernel(x, tok_table, pos_table):
  x_flat = jnp.reshape(x.astype(jnp.int32), (_B * _L,))
  tok_rows = jnp.reshape(
      jnp.pad(tok_table, ((0, 0), (0, _HID))), (2 * _VOCAB, _HID))
  out_wide = _tpe_call(x_flat, tok_rows, pos_table)
  return out_wide[:, :, :_HID]
